# baseline (speedup 1.0000x reference)
import functools

import jax
import jax.numpy as jnp
from jax import lax
from jax.experimental import pallas as pl
from jax.experimental.pallas import tpu as pltpu
from jax.experimental.pallas import tpu_sc as plsc

B_USERS = 256
B_ITEMS = 256
HIDDEN_DIM = 128
N_ROWS = 1024


def _sc_body(uid_hbm, out_hbm, idx_v):
  @pl.when(lax.axis_index("s") + lax.axis_index("c") == 0)
  def _():
    pltpu.sync_copy(uid_hbm.at[pl.ds(0, 16)], idx_v)
    pltpu.sync_copy(idx_v, out_hbm.at[pl.ds(0, 16)])


_sc_probe = functools.partial(
    pl.kernel,
    out_type=jax.ShapeDtypeStruct((16,), jnp.int32),
    mesh=plsc.VectorSubcoreMesh(core_axis_name="c", subcore_axis_name="s"),
    scratch_types=[pltpu.VMEM((16,), jnp.int32)],
)(_sc_body)


def _body(uid_ref, iid_ref, utab_ref, itab_ref, o_ref):
  uid = uid_ref[0]
  iid = iid_ref[0]
  rows = lax.broadcasted_iota(jnp.int32, (B_USERS, N_ROWS), 1)
  pu = (uid[:, None] == rows).astype(jnp.float32)
  pv = (iid[:, None] == rows).astype(jnp.float32)
  u = jnp.dot(pu, utab_ref[...], preferred_element_type=jnp.float32)
  v = jnp.dot(pv, itab_ref[...], preferred_element_type=jnp.float32)
  o_ref[...] = lax.dot_general(
      u, v, dimension_numbers=(((1,), (1,)), ((), ())),
      preferred_element_type=jnp.float32)


_call = pl.pallas_call(
    _body,
    out_shape=jax.ShapeDtypeStruct((B_USERS, B_ITEMS), jnp.float32),
)


@jax.jit
def kernel(user_ids, item_ids, user_table, item_table):
  probe = _sc_probe(user_ids)
  out = _call(user_ids.reshape(1, B_USERS), item_ids.reshape(1, B_ITEMS),
              user_table, item_table)
  return out + jnp.float32(0.0) * probe[0].astype(jnp.float32)


# speedup vs baseline: 7.3148x; 7.3148x over previous
import jax
import jax.numpy as jnp
from jax import lax
from jax.experimental import pallas as pl

B_USERS = 256
B_ITEMS = 256
HIDDEN_DIM = 128
N_ROWS = 1024


def _split_dot(p, tab):
  hi = tab.astype(jnp.bfloat16)
  lo = (tab - hi.astype(jnp.float32)).astype(jnp.bfloat16)
  out = jnp.dot(p, hi, preferred_element_type=jnp.float32)
  out += jnp.dot(p, lo, preferred_element_type=jnp.float32)
  return out


def _body(uid_ref, iid_ref, utab_ref, itab_ref, o_ref):
  uid = uid_ref[0]
  iid = iid_ref[0]
  rows = lax.broadcasted_iota(jnp.int32, (B_USERS, N_ROWS), 1)
  pu = (uid[:, None] == rows).astype(jnp.bfloat16)
  pv = (iid[:, None] == rows).astype(jnp.bfloat16)
  u = _split_dot(pu, utab_ref[...])
  v = _split_dot(pv, itab_ref[...])
  o_ref[...] = lax.dot_general(
      u, v, dimension_numbers=(((1,), (1,)), ((), ())),
      preferred_element_type=jnp.float32)


_call = pl.pallas_call(
    _body,
    out_shape=jax.ShapeDtypeStruct((B_USERS, B_ITEMS), jnp.float32),
)


@jax.jit
def kernel(user_ids, item_ids, user_table, item_table):
  return _call(user_ids.reshape(1, B_USERS), item_ids.reshape(1, B_ITEMS),
               user_table, item_table)
